# Initial kernel scaffold; baseline (speedup 1.0000x reference)
import functools

import jax
import jax.numpy as jnp
from jax import lax
from jax.experimental import pallas as pl
from jax.experimental.pallas import tpu as pltpu
from jax.experimental.pallas import tpu_sc as plsc

EMBED_DIM = 32
NUM_CORES = 2
NUM_SUBCORES = 16
NUM_WORKERS = NUM_CORES * NUM_SUBCORES

CHUNK = 1600


def _build_gather(total_rows: int):
    assert total_rows % NUM_WORKERS == 0
    rows_per_worker = total_rows // NUM_WORKERS
    assert rows_per_worker % CHUNK == 0
    num_chunks = rows_per_worker // CHUNK

    mesh = plsc.VectorSubcoreMesh(core_axis_name="c", subcore_axis_name="s")

    @functools.partial(
        pl.kernel,
        mesh=mesh,
        out_type=jax.ShapeDtypeStruct((total_rows, EMBED_DIM), jnp.float32),
        scratch_types=[
            pltpu.VMEM((CHUNK,), jnp.int32),
            pltpu.VMEM((CHUNK, EMBED_DIM), jnp.float32),
            pltpu.SemaphoreType.DMA,
        ],
    )
    def gather_kernel(idx_hbm, table_hbm, out_hbm, idx_v, rows_v, sem):
        wid = lax.axis_index("s") * NUM_CORES + lax.axis_index("c")
        base = wid * rows_per_worker

        def chunk_body(i, carry):
            off = base + i * CHUNK
            pltpu.sync_copy(idx_hbm.at[pl.ds(off, CHUNK)], idx_v)
            pltpu.async_copy(table_hbm.at[idx_v], rows_v, sem).wait()
            pltpu.sync_copy(rows_v, out_hbm.at[pl.ds(off, CHUNK)])
            return carry

        lax.fori_loop(0, num_chunks, chunk_body, 0)

    return gather_kernel


def kernel(x, table):
    b0, b1 = x.shape
    flat_idx = x.reshape(-1).astype(jnp.int32)
    gathered = _build_gather(b0 * b1)(flat_idx, table)
    return gathered.reshape(b0, b1, EMBED_DIM)


# speedup vs baseline: 1.4813x; 1.4813x over previous
import functools

import jax
import jax.numpy as jnp
from jax import lax
from jax.experimental import pallas as pl
from jax.experimental.pallas import tpu as pltpu
from jax.experimental.pallas import tpu_sc as plsc

EMBED_DIM = 32
NUM_CORES = 2
NUM_SUBCORES = 16
NUM_WORKERS = NUM_CORES * NUM_SUBCORES

CHUNK = 1600


def _build_gather(total_rows: int):
    assert total_rows % NUM_WORKERS == 0
    rows_per_worker = total_rows // NUM_WORKERS
    assert rows_per_worker % CHUNK == 0
    num_chunks = rows_per_worker // CHUNK

    mesh = plsc.VectorSubcoreMesh(core_axis_name="c", subcore_axis_name="s")

    @functools.partial(
        pl.kernel,
        mesh=mesh,
        out_type=jax.ShapeDtypeStruct((total_rows, EMBED_DIM), jnp.float32),
        scratch_types=[
            pltpu.VMEM((CHUNK,), jnp.int32),
            pltpu.VMEM((CHUNK, EMBED_DIM), jnp.float32),
            pltpu.SemaphoreType.DMA,
        ],
        compiler_params=pltpu.CompilerParams(use_tc_tiling_on_sc=False),
    )
    def gather_kernel(idx_hbm, table_hbm, out_hbm, idx_v, rows_v, sem):
        wid = lax.axis_index("s") * NUM_CORES + lax.axis_index("c")
        base = wid * rows_per_worker

        def chunk_body(i, carry):
            off = base + i * CHUNK
            pltpu.sync_copy(idx_hbm.at[pl.ds(off, CHUNK)], idx_v)
            pltpu.async_copy(table_hbm.at[idx_v], rows_v, sem).wait()
            pltpu.sync_copy(rows_v, out_hbm.at[pl.ds(off, CHUNK)])
            return carry

        lax.fori_loop(0, num_chunks, chunk_body, 0)

    return gather_kernel


def kernel(x, table):
    b0, b1 = x.shape
    flat_idx = x.reshape(-1).astype(jnp.int32)
    gathered = _build_gather(b0 * b1)(flat_idx, table)
    return gathered.reshape(b0, b1, EMBED_DIM)


# speedup vs baseline: 1.4981x; 1.0113x over previous
import functools

import jax
import jax.numpy as jnp
from jax import lax
from jax.experimental import pallas as pl
from jax.experimental.pallas import tpu as pltpu
from jax.experimental.pallas import tpu_sc as plsc

EMBED_DIM = 32
NUM_CORES = 2
NUM_SUBCORES = 16
NUM_WORKERS = NUM_CORES * NUM_SUBCORES

CHUNK = 800
K = 4


def _build_gather(total_rows: int):
    assert total_rows % NUM_WORKERS == 0
    rows_per_worker = total_rows // NUM_WORKERS
    assert rows_per_worker % (CHUNK * K) == 0
    num_groups = rows_per_worker // (CHUNK * K)

    mesh = plsc.VectorSubcoreMesh(core_axis_name="c", subcore_axis_name="s")

    scratch = (
        [pltpu.VMEM((CHUNK,), jnp.int32) for _ in range(K)]
        + [pltpu.VMEM((CHUNK, EMBED_DIM), jnp.float32) for _ in range(K)]
        + [pltpu.SemaphoreType.DMA for _ in range(2 * K)]
    )

    @functools.partial(
        pl.kernel,
        mesh=mesh,
        out_type=jax.ShapeDtypeStruct((total_rows, EMBED_DIM), jnp.float32),
        scratch_types=scratch,
        compiler_params=pltpu.CompilerParams(use_tc_tiling_on_sc=False),
    )
    def gather_kernel(idx_hbm, table_hbm, out_hbm, *bufs):
        idx_v = bufs[:K]
        rows_v = bufs[K:2 * K]
        gsem = bufs[2 * K:3 * K]
        osem = bufs[3 * K:4 * K]

        wid = lax.axis_index("s") * NUM_CORES + lax.axis_index("c")
        base = wid * rows_per_worker

        def group_body(g, carry):
            goff = base + g * (CHUNK * K)
            for b in range(K):
                off = goff + b * CHUNK
                pltpu.sync_copy(idx_hbm.at[pl.ds(off, CHUNK)], idx_v[b])
                pltpu.async_copy(table_hbm.at[idx_v[b]], rows_v[b], gsem[b])
            for b in range(K):
                off = goff + b * CHUNK
                pltpu.make_async_copy(
                    table_hbm.at[idx_v[b]], rows_v[b], gsem[b]).wait()
                pltpu.async_copy(rows_v[b], out_hbm.at[pl.ds(off, CHUNK)],
                                 osem[b])
            for b in range(K):
                off = goff + b * CHUNK
                pltpu.make_async_copy(
                    rows_v[b], out_hbm.at[pl.ds(off, CHUNK)], osem[b]).wait()
            return carry

        lax.fori_loop(0, num_groups, group_body, 0)

    return gather_kernel


def kernel(x, table):
    b0, b1 = x.shape
    flat_idx = x.reshape(-1).astype(jnp.int32)
    gathered = _build_gather(b0 * b1)(flat_idx, table)
    return gathered.reshape(b0, b1, EMBED_DIM)
